# baseline (speedup 1.0000x reference)
import functools

import jax
import jax.numpy as jnp
from jax import lax
from jax.experimental import pallas as pl
from jax.experimental.pallas import tpu as pltpu
from jax.experimental.pallas import tpu_sc as plsc

N_E = 8192
E_DIM = 256
N_TOK = 8192
ROWS = 256
GRID = N_TOK // ROWS

NW = 32
B_PER_W = N_TOK // NW
CHUNK = 128


B0 = 4096


def _dist_argmin_body(z_ref, embt_ref, zs_ref, es_ref, oh_ref, idx_ref, dmin_ref):
    z_blk = z_ref[...]
    mm = lax.dot_general(z_blk, embt_ref[...],
                         (((1,), (0,)), ((), ())),
                         preferred_element_type=jnp.float32)
    dmat = (zs_ref[...] + es_ref[...]) - 2.0 * mm
    cols = lax.broadcasted_iota(jnp.int32, (ROWS, N_E), 1)
    chalf = lax.broadcasted_iota(jnp.int32, (ROWS, B0), 1)

    d0 = dmat[:, :B0]
    d1 = dmat[:, B0:]
    m0 = jnp.min(d0, axis=1)
    m1 = jnp.min(d1, axis=1)
    a0 = jnp.min(jnp.where(d0 == m0[:, None], chalf, N_E), axis=1)
    a1 = jnp.min(jnp.where(d1 == m1[:, None], chalf, N_E), axis=1) + B0
    run = m0.astype(jnp.bfloat16).astype(jnp.float32)
    upd = m1 < run
    idx = jnp.where(upd, a1, a0)
    d_sel = jnp.where(upd, m1, m0)

    oh_ref[...] = (cols == idx[:, None]).astype(jnp.float32)
    idx_ref[...] = idx
    dmin_ref[...] = d_sel.reshape(1, 1, ROWS)


def _sc_gather_hist_body(emb_hbm, idx_hbm, zq_hbm, counts_hbm,
                         idx_v, rows_v, ones_v, zeros_v, hist_sh, sem):
    c = lax.axis_index("c")
    s = lax.axis_index("s")
    wid = s * 2 + c
    base = wid * B_PER_W
    one16 = jnp.full((16,), 1.0, jnp.float32)
    zero16 = jnp.zeros((16,), jnp.float32)
    for i in range(CHUNK // 16):
        ones_v[pl.ds(i * 16, 16)] = one16
    for i in range((N_E // 16) // 16):
        zeros_v[pl.ds(i * 16, 16)] = zero16
    pltpu.sync_copy(idx_hbm.at[pl.ds(wid * 2, 2)], idx_v)
    pltpu.sync_copy(zeros_v, hist_sh.at[pl.ds(s * (N_E // 16), N_E // 16)])
    plsc.subcore_barrier()
    for j in range(B_PER_W // CHUNK):
        pltpu.async_copy(emb_hbm.at[idx_v.at[j]], rows_v, sem).wait()
        pltpu.sync_copy(rows_v, zq_hbm.at[pl.ds(base + j * CHUNK, CHUNK)])
        pltpu.sync_copy(ones_v, hist_sh.at[idx_v.at[j]], add=True)
    plsc.subcore_barrier()

    @pl.when(s == 0)
    def _():
        pltpu.sync_copy(hist_sh, counts_hbm.at[c])


def _finalize_body(counts_ref, dmin_ref, loss_ref, perp_ref):
    counts = counts_ref[0, :] + counts_ref[1, :]
    e_mean = counts * (1.0 / N_TOK)
    ent = -jnp.sum(e_mean * jnp.log(e_mean + 1e-10))
    perp_ref[...] = jnp.exp(ent).reshape(1, 1)
    loss_ref[...] = (1.25 * jnp.sum(dmin_ref[...]) / (N_TOK * E_DIM)).reshape(1, 1)


def kernel(z, emb):
    z_flat = z.reshape(-1, E_DIM)
    zs = jnp.sum(z_flat ** 2, axis=1, keepdims=True)
    es = jnp.sum(emb ** 2, axis=1)[None, :]
    embt = emb.T

    onehot, idx3, dmin3 = pl.pallas_call(
        _dist_argmin_body,
        grid=(GRID,),
        in_specs=[
            pl.BlockSpec((ROWS, E_DIM), lambda i: (i, 0)),
            pl.BlockSpec((E_DIM, N_E), lambda i: (0, 0)),
            pl.BlockSpec((ROWS, 1), lambda i: (i, 0)),
            pl.BlockSpec((1, N_E), lambda i: (0, 0)),
        ],
        out_specs=[
            pl.BlockSpec((ROWS, N_E), lambda i: (i, 0)),
            pl.BlockSpec((ROWS,), lambda i: (i,)),
            pl.BlockSpec((1, 1, ROWS), lambda i: (i, 0, 0)),
        ],
        out_shape=[
            jax.ShapeDtypeStruct((N_TOK, N_E), jnp.float32),
            jax.ShapeDtypeStruct((N_TOK,), jnp.int32),
            jax.ShapeDtypeStruct((GRID, 1, ROWS), jnp.float32),
        ],
    )(z_flat, embt, zs, es)

    idx_flat = idx3
    idx2d = idx_flat.reshape(N_TOK // CHUNK, CHUNK)

    mesh = plsc.VectorSubcoreMesh(core_axis_name="c", subcore_axis_name="s")
    zq_flat, counts2 = pl.kernel(
        _sc_gather_hist_body,
        out_type=[
            jax.ShapeDtypeStruct((N_TOK, E_DIM), jnp.float32),
            jax.ShapeDtypeStruct((2, N_E), jnp.float32),
        ],
        mesh=mesh,
        scratch_types=[
            pltpu.VMEM((2, CHUNK), jnp.int32),
            pltpu.VMEM((CHUNK, E_DIM), jnp.float32),
            pltpu.VMEM((CHUNK,), jnp.float32),
            pltpu.VMEM((N_E // 16,), jnp.float32),
            pltpu.VMEM_SHARED((N_E,), jnp.float32),
            pltpu.SemaphoreType.DMA,
        ],
    )(emb, idx2d)

    loss1, perp1 = pl.pallas_call(
        _finalize_body,
        in_specs=[
            pl.BlockSpec((2, N_E), lambda: (0, 0)),
            pl.BlockSpec((GRID, 1, ROWS), lambda: (0, 0, 0)),
        ],
        out_specs=[
            pl.BlockSpec((1, 1), lambda: (0, 0)),
            pl.BlockSpec((1, 1), lambda: (0, 0)),
        ],
        out_shape=[
            jax.ShapeDtypeStruct((1, 1), jnp.float32),
            jax.ShapeDtypeStruct((1, 1), jnp.float32),
        ],
    )(counts2, dmin3)

    loss = loss1.reshape(())
    perplexity = perp1.reshape(())
    z_q_st = zq_flat.reshape(z.shape)
    return (loss, z_q_st, perplexity, onehot, idx_flat[:, None])


# speedup vs baseline: 1.0303x; 1.0303x over previous
import functools

import jax
import jax.numpy as jnp
from jax import lax
from jax.experimental import pallas as pl
from jax.experimental.pallas import tpu as pltpu
from jax.experimental.pallas import tpu_sc as plsc

N_E = 8192
E_DIM = 256
N_TOK = 8192
ROWS = 512
GRID = N_TOK // ROWS

NW = 32
B_PER_W = N_TOK // NW
CHUNK = 128


B0 = 4096


def _dist_argmin_body(z_ref, embt_ref, zs_ref, es_ref, oh_ref, idx_ref, dmin_ref):
    z_blk = z_ref[...]
    mm = lax.dot_general(z_blk, embt_ref[...],
                         (((1,), (0,)), ((), ())),
                         preferred_element_type=jnp.float32)
    dmat = (zs_ref[...] + es_ref[...]) - 2.0 * mm
    cols = lax.broadcasted_iota(jnp.int32, (ROWS, N_E), 1)
    chalf = lax.broadcasted_iota(jnp.int32, (ROWS, B0), 1)

    d0 = dmat[:, :B0]
    d1 = dmat[:, B0:]
    m0 = jnp.min(d0, axis=1)
    m1 = jnp.min(d1, axis=1)
    a0 = jnp.min(jnp.where(d0 == m0[:, None], chalf, N_E), axis=1)
    a1 = jnp.min(jnp.where(d1 == m1[:, None], chalf, N_E), axis=1) + B0
    run = m0.astype(jnp.bfloat16).astype(jnp.float32)
    upd = m1 < run
    idx = jnp.where(upd, a1, a0)
    d_sel = jnp.where(upd, m1, m0)

    oh_ref[...] = (cols == idx[:, None]).astype(jnp.float32)
    idx_ref[...] = idx
    dmin_ref[...] = d_sel.reshape(1, 1, ROWS)


def _sc_gather_hist_body(emb_hbm, idx_hbm, zq_hbm, counts_hbm,
                         idx_v, rows_v, ones_v, zeros_v, hist_sh, sem):
    c = lax.axis_index("c")
    s = lax.axis_index("s")
    wid = s * 2 + c
    base = wid * B_PER_W
    one16 = jnp.full((16,), 1.0, jnp.float32)
    zero16 = jnp.zeros((16,), jnp.float32)
    for i in range(CHUNK // 16):
        ones_v[pl.ds(i * 16, 16)] = one16
    for i in range((N_E // 16) // 16):
        zeros_v[pl.ds(i * 16, 16)] = zero16
    pltpu.sync_copy(idx_hbm.at[pl.ds(wid * 2, 2)], idx_v)
    pltpu.sync_copy(zeros_v, hist_sh.at[pl.ds(s * (N_E // 16), N_E // 16)])
    plsc.subcore_barrier()
    for j in range(B_PER_W // CHUNK):
        pltpu.async_copy(emb_hbm.at[idx_v.at[j]], rows_v, sem).wait()
        pltpu.sync_copy(rows_v, zq_hbm.at[pl.ds(base + j * CHUNK, CHUNK)])
        pltpu.sync_copy(ones_v, hist_sh.at[idx_v.at[j]], add=True)
    plsc.subcore_barrier()

    @pl.when(s == 0)
    def _():
        pltpu.sync_copy(hist_sh, counts_hbm.at[c])


def _finalize_body(counts_ref, dmin_ref, loss_ref, perp_ref):
    counts = counts_ref[0, :] + counts_ref[1, :]
    e_mean = counts * (1.0 / N_TOK)
    ent = -jnp.sum(e_mean * jnp.log(e_mean + 1e-10))
    perp_ref[...] = jnp.exp(ent).reshape(1, 1)
    loss_ref[...] = (1.25 * jnp.sum(dmin_ref[...]) / (N_TOK * E_DIM)).reshape(1, 1)


def kernel(z, emb):
    z_flat = z.reshape(-1, E_DIM)
    zs = jnp.sum(z_flat ** 2, axis=1, keepdims=True)
    es = jnp.sum(emb ** 2, axis=1)[None, :]
    embt = emb.T

    onehot, idx3, dmin3 = pl.pallas_call(
        _dist_argmin_body,
        grid=(GRID,),
        in_specs=[
            pl.BlockSpec((ROWS, E_DIM), lambda i: (i, 0)),
            pl.BlockSpec((E_DIM, N_E), lambda i: (0, 0)),
            pl.BlockSpec((ROWS, 1), lambda i: (i, 0)),
            pl.BlockSpec((1, N_E), lambda i: (0, 0)),
        ],
        out_specs=[
            pl.BlockSpec((ROWS, N_E), lambda i: (i, 0)),
            pl.BlockSpec((ROWS,), lambda i: (i,)),
            pl.BlockSpec((1, 1, ROWS), lambda i: (i, 0, 0)),
        ],
        out_shape=[
            jax.ShapeDtypeStruct((N_TOK, N_E), jnp.float32),
            jax.ShapeDtypeStruct((N_TOK,), jnp.int32),
            jax.ShapeDtypeStruct((GRID, 1, ROWS), jnp.float32),
        ],
    )(z_flat, embt, zs, es)

    idx_flat = idx3
    idx2d = idx_flat.reshape(N_TOK // CHUNK, CHUNK)

    mesh = plsc.VectorSubcoreMesh(core_axis_name="c", subcore_axis_name="s")
    zq_flat, counts2 = pl.kernel(
        _sc_gather_hist_body,
        out_type=[
            jax.ShapeDtypeStruct((N_TOK, E_DIM), jnp.float32),
            jax.ShapeDtypeStruct((2, N_E), jnp.float32),
        ],
        mesh=mesh,
        scratch_types=[
            pltpu.VMEM((2, CHUNK), jnp.int32),
            pltpu.VMEM((CHUNK, E_DIM), jnp.float32),
            pltpu.VMEM((CHUNK,), jnp.float32),
            pltpu.VMEM((N_E // 16,), jnp.float32),
            pltpu.VMEM_SHARED((N_E,), jnp.float32),
            pltpu.SemaphoreType.DMA,
        ],
    )(emb, idx2d)

    loss1, perp1 = pl.pallas_call(
        _finalize_body,
        in_specs=[
            pl.BlockSpec((2, N_E), lambda: (0, 0)),
            pl.BlockSpec((GRID, 1, ROWS), lambda: (0, 0, 0)),
        ],
        out_specs=[
            pl.BlockSpec((1, 1), lambda: (0, 0)),
            pl.BlockSpec((1, 1), lambda: (0, 0)),
        ],
        out_shape=[
            jax.ShapeDtypeStruct((1, 1), jnp.float32),
            jax.ShapeDtypeStruct((1, 1), jnp.float32),
        ],
    )(counts2, dmin3)

    loss = loss1.reshape(())
    perplexity = perp1.reshape(())
    z_q_st = zq_flat.reshape(z.shape)
    return (loss, z_q_st, perplexity, onehot, idx_flat[:, None])
